# Initial kernel scaffold; baseline (speedup 1.0000x reference)
import jax
import jax.numpy as jnp
from jax.experimental import pallas as pl


def kernel(connections, num_neighbors, symbol_emb, W_gcn, b_gcn, gcn_b, W_attn, b_attn, W_gate, b_gate, gate_b):
    raise NotImplementedError("write your pallas kernel here")


# speedup vs baseline: 13.6098x; 13.6098x over previous
import functools
import jax
import jax.numpy as jnp
from jax.experimental import pallas as pl
from jax.experimental.pallas import tpu as pltpu

B = 4096
NB = 200
D = 100
BLK = 64


def _body(rel_e, ent_e, nb, selfe, u12, w1, w2, bias, wg, cvec, out):
    c = cvec[0, 0]
    bg = cvec[0, 1]
    r2 = rel_e[...].reshape(BLK * NB, D)
    e2 = ent_e[...].reshape(BLK * NB, D)
    lg = jnp.sum(r2 * u12[0:1, :] + e2 * u12[1:2, :], axis=1, keepdims=True)
    logit = lg.reshape(BLK, NB) + c
    logit = jnp.where(logit > 0, logit, 0.2 * logit)
    nbm = jnp.maximum(nb[...], 1)
    mask = jax.lax.broadcasted_iota(jnp.int32, (BLK, NB), 1) < nbm
    logit = jnp.where(mask, logit, -1e9)
    m = jnp.max(logit, axis=1, keepdims=True)
    w = jnp.exp(logit - m)
    w = w / jnp.sum(w, axis=1, keepdims=True)
    r_sum = jax.lax.dot_general(w, rel_e[...], (((1,), (1,)), ((0,), (0,))),
                                preferred_element_type=jnp.float32)
    e_sum = jax.lax.dot_general(w, ent_e[...], (((1,), (1,)), ((0,), (0,))),
                                preferred_element_type=jnp.float32)
    agg = (jnp.dot(r_sum, w1[...].T, preferred_element_type=jnp.float32)
           + jnp.dot(e_sum, w2[...].T, preferred_element_type=jnp.float32) + bias[0])
    glog = jnp.sum(agg * wg[...], axis=1, keepdims=True)
    g = jax.nn.sigmoid(glog + bg)
    out[...] = jnp.tanh(g * agg + (1.0 - g) * selfe[...])


def kernel(connections, num_neighbors, symbol_emb, W_gcn, b_gcn, gcn_b, W_attn, b_attn, W_gate, b_gate, gate_b):
    relations = connections[:, :, 1]
    entities = connections[:, :, 2]
    entself = connections[:, 0, 0]
    rel_e = jnp.take(symbol_emb, relations, axis=0)
    ent_e = jnp.take(symbol_emb, entities, axis=0)
    self_e = jnp.take(symbol_emb, entself, axis=0)
    u12 = (W_attn @ W_gcn).reshape(2, D)
    c = W_attn[0] @ (b_gcn + gcn_b) + b_attn[0]
    bg = b_gate[0] + gate_b[0]
    cvec = jnp.stack([c, bg])[None, :]
    bias = (b_gcn + gcn_b)[None, :]
    nb2 = num_neighbors[:, None].astype(jnp.int32)

    grid = B // BLK
    out = pl.pallas_call(
        _body,
        grid=(grid,),
        in_specs=[
            pl.BlockSpec((BLK, NB, D), lambda i: (i, 0, 0)),
            pl.BlockSpec((BLK, NB, D), lambda i: (i, 0, 0)),
            pl.BlockSpec((BLK, 1), lambda i: (i, 0)),
            pl.BlockSpec((BLK, D), lambda i: (i, 0)),
            pl.BlockSpec((2, D), lambda i: (0, 0)),
            pl.BlockSpec((D, D), lambda i: (0, 0)),
            pl.BlockSpec((D, D), lambda i: (0, 0)),
            pl.BlockSpec((1, D), lambda i: (0, 0)),
            pl.BlockSpec((1, D), lambda i: (0, 0)),
            pl.BlockSpec((1, 2), lambda i: (0, 0)),
        ],
        out_specs=pl.BlockSpec((BLK, D), lambda i: (i, 0)),
        out_shape=jax.ShapeDtypeStruct((B, D), jnp.float32),
    )(rel_e, ent_e, nb2, self_e, u12, W_gcn[:, :D], W_gcn[:, D:], bias, W_gate, cvec)
    return out
